# baseline (speedup 1.0000x reference)
import functools

import jax
import jax.numpy as jnp
from jax import lax
from jax.experimental import pallas as pl
from jax.experimental.pallas import tpu as pltpu
from jax.experimental.pallas import tpu_sc as plsc

NC = 2
NS = 16
NW = NC * NS
LANES = 16

D = 1024
SEQ = 2048
ROWS = 4 * SEQ
RPW = ROWS // NW
CH = 32
NCHUNK = RPW // CH


def _make_sc_kernel():
    mesh = plsc.VectorSubcoreMesh(core_axis_name="c", subcore_axis_name="s")

    @functools.partial(
        pl.kernel,
        mesh=mesh,
        out_type=jax.ShapeDtypeStruct((ROWS, D), jnp.float32),
        scratch_types=[
            pltpu.VMEM((RPW,), jnp.int32),
            pltpu.VMEM((CH, D), jnp.float32),
            pltpu.VMEM((CH, D), jnp.float32),
            pltpu.SemaphoreType.DMA,
        ],
    )
    def emb_kernel(x_hbm, wte_hbm, wpe_hbm, out_hbm, idx_v, rows_v, pos_v, sem):
        wid = lax.axis_index("s") * NC + lax.axis_index("c")
        base = wid * RPW
        sbase = lax.rem(base, SEQ)

        pltpu.sync_copy(x_hbm.at[pl.ds(base, RPW)], idx_v)

        def chunk_body(c, carry):
            rbase = c * CH
            pltpu.async_copy(
                wte_hbm.at[idx_v.at[pl.ds(rbase, CH)]], rows_v, sem
            ).wait()
            pltpu.sync_copy(wpe_hbm.at[pl.ds(sbase + rbase, CH)], pos_v)

            def row_body(i, rcarry):
                for j in range(D // LANES):
                    sl = pl.ds(j * LANES, LANES)
                    rows_v[i, sl] = rows_v[i, sl] + pos_v[i, sl]
                return rcarry

            lax.fori_loop(0, CH, row_body, 0)
            pltpu.sync_copy(rows_v, out_hbm.at[pl.ds(base + rbase, CH)])
            return carry

        lax.fori_loop(0, NCHUNK, chunk_body, 0)

    return emb_kernel


_SC_KERNEL = _make_sc_kernel()


def kernel(x, wte, wpe):
    b, s = x.shape
    xf = x.reshape(-1).astype(jnp.int32)
    out = _SC_KERNEL(xf, wte, wpe)
    return out.reshape(b, s, D)


# speedup vs baseline: 1.0808x; 1.0808x over previous
import functools

import jax
import jax.numpy as jnp
from jax import lax
from jax.experimental import pallas as pl
from jax.experimental.pallas import tpu as pltpu
from jax.experimental.pallas import tpu_sc as plsc

NC = 2
NS = 16
NW = NC * NS
LANES = 16

D = 1024
SEQ = 2048
B = 4
ROWS = B * SEQ
PPW = SEQ // NW
CH = 16
CPB = PPW // CH
NCHUNK = B * CPB


def _make_sc_kernel():
    mesh = plsc.VectorSubcoreMesh(core_axis_name="c", subcore_axis_name="s")

    @functools.partial(
        pl.kernel,
        mesh=mesh,
        out_type=jax.ShapeDtypeStruct((ROWS, D), jnp.float32),
        scratch_types=[
            pltpu.VMEM((B * PPW,), jnp.int32),
            pltpu.VMEM((PPW, D), jnp.float32),
            pltpu.VMEM((CH, D), jnp.float32),
            pltpu.VMEM((CH, D), jnp.float32),
            pltpu.SemaphoreType.DMA,
            pltpu.SemaphoreType.DMA,
            pltpu.SemaphoreType.DMA,
            pltpu.SemaphoreType.DMA,
        ],
    )
    def emb_kernel(x_hbm, wte_hbm, wpe_hbm, out_hbm, idx_v, wpe_v,
                   rows0, rows1, gsem0, gsem1, ssem0, ssem1):
        wid = lax.axis_index("s") * NC + lax.axis_index("c")
        poff = wid * PPW

        pltpu.sync_copy(wpe_hbm.at[pl.ds(poff, PPW)], wpe_v)
        for b in range(B):
            pltpu.sync_copy(
                x_hbm.at[pl.ds(b * SEQ + poff, PPW)],
                idx_v.at[pl.ds(b * PPW, PPW)],
            )

        bufs = (rows0, rows1)
        gsems = (gsem0, gsem1)
        ssems = (ssem0, ssem1)
        gather_h = [None, None]
        store_h = [None, None]

        def start_gather(c):
            k = c % 2
            if store_h[k] is not None:
                store_h[k].wait()
                store_h[k] = None
            gather_h[k] = pltpu.async_copy(
                wte_hbm.at[idx_v.at[pl.ds(c * CH, CH)]], bufs[k], gsems[k]
            )

        def finish_chunk(c):
            k = c % 2
            gather_h[k].wait()
            qi = c % CPB
            rows = bufs[k]

            def row_body(i, carry):
                for j in range(D // LANES):
                    sl = pl.ds(j * LANES, LANES)
                    rows[i, sl] = rows[i, sl] + wpe_v[qi * CH + i, sl]
                return carry

            lax.fori_loop(0, CH, row_body, 0)
            b = c // CPB
            store_h[k] = pltpu.async_copy(
                rows, out_hbm.at[pl.ds(b * SEQ + poff + qi * CH, CH)], ssems[k]
            )

        start_gather(0)
        for c in range(NCHUNK):
            if c + 1 < NCHUNK:
                start_gather(c + 1)
            finish_chunk(c)
        store_h[0].wait()
        store_h[1].wait()

    return emb_kernel


_SC_KERNEL = _make_sc_kernel()


def kernel(x, wte, wpe):
    b, s = x.shape
    xf = x.reshape(-1).astype(jnp.int32)
    out = _SC_KERNEL(xf, wte, wpe)
    return out.reshape(b, s, D)


# speedup vs baseline: 1.0824x; 1.0015x over previous
import functools

import jax
import jax.numpy as jnp
from jax import lax
from jax.experimental import pallas as pl
from jax.experimental.pallas import tpu as pltpu
from jax.experimental.pallas import tpu_sc as plsc

NC = 2
NS = 16
NW = NC * NS
LANES = 16

D = 1024
SEQ = 2048
B = 4
ROWS = B * SEQ
PPW = SEQ // NW
CH = 16
CPB = PPW // CH
NCHUNK = B * CPB


def _make_sc_kernel():
    mesh = plsc.VectorSubcoreMesh(core_axis_name="c", subcore_axis_name="s")

    @functools.partial(
        pl.kernel,
        mesh=mesh,
        out_type=jax.ShapeDtypeStruct((ROWS, D), jnp.float32),
        scratch_types=[
            pltpu.VMEM((B * PPW,), jnp.int32),
            pltpu.VMEM((PPW, D), jnp.float32),
            pltpu.VMEM((CH, D), jnp.float32),
            pltpu.VMEM((CH, D), jnp.float32),
            pltpu.VMEM((CH, D), jnp.float32),
            pltpu.SemaphoreType.DMA,
            pltpu.SemaphoreType.DMA,
            pltpu.SemaphoreType.DMA,
            pltpu.SemaphoreType.DMA,
            pltpu.SemaphoreType.DMA,
            pltpu.SemaphoreType.DMA,
        ],
    )
    def emb_kernel(x_hbm, wte_hbm, wpe_hbm, out_hbm, idx_v, wpe_v,
                   rows0, rows1, rows2, gsem0, gsem1, gsem2,
                   ssem0, ssem1, ssem2):
        wid = lax.axis_index("s") * NC + lax.axis_index("c")
        poff = wid * PPW

        pltpu.sync_copy(wpe_hbm.at[pl.ds(poff, PPW)], wpe_v)
        for b in range(B):
            pltpu.sync_copy(
                x_hbm.at[pl.ds(b * SEQ + poff, PPW)],
                idx_v.at[pl.ds(b * PPW, PPW)],
            )

        NBUF = 3
        bufs = (rows0, rows1, rows2)
        gsems = (gsem0, gsem1, gsem2)
        ssems = (ssem0, ssem1, ssem2)
        gather_h = [None] * NBUF
        store_h = [None] * NBUF

        def start_gather(c):
            k = c % NBUF
            if store_h[k] is not None:
                store_h[k].wait()
                store_h[k] = None
            gather_h[k] = pltpu.async_copy(
                wte_hbm.at[idx_v.at[pl.ds(c * CH, CH)]], bufs[k], gsems[k]
            )

        def finish_chunk(c):
            k = c % NBUF
            gather_h[k].wait()
            qi = c % CPB
            rows = bufs[k]

            def row_body(i, carry):
                for j in range(D // LANES):
                    sl = pl.ds(j * LANES, LANES)
                    rows[i, sl] = rows[i, sl] + wpe_v[qi * CH + i, sl]
                return carry

            lax.fori_loop(0, CH, row_body, 0)
            b = c // CPB
            store_h[k] = pltpu.async_copy(
                rows, out_hbm.at[pl.ds(b * SEQ + poff + qi * CH, CH)], ssems[k]
            )

        for c in range(NBUF - 1):
            start_gather(c)
        for c in range(NCHUNK):
            if c + NBUF - 1 < NCHUNK:
                start_gather(c + NBUF - 1)
            finish_chunk(c)
        for k in range(NBUF):
            if store_h[k] is not None:
                store_h[k].wait()

    return emb_kernel


_SC_KERNEL = _make_sc_kernel()


def kernel(x, wte, wpe):
    b, s = x.shape
    xf = x.reshape(-1).astype(jnp.int32)
    out = _SC_KERNEL(xf, wte, wpe)
    return out.reshape(b, s, D)
